# baseline (speedup 1.0000x reference)
import functools

import jax
import jax.numpy as jnp
from jax import lax
from jax.experimental import pallas as pl
from jax.experimental.pallas import tpu as pltpu
from jax.experimental.pallas import tpu_sc as plsc

N, E, D, H = 10000, 160000, 256, 512
QD = 64
NQ = 4
NC = 2
NT = 16
EP = 163840
EPT = EP // NT
KB = 256
M = EPT // KB
NP = 10240
RPT = NP // NT
DUMP = 10100


def _sc_aggregate(xq4, gidx4, ddst, oidx, eyeb, za):
    mesh = plsc.VectorSubcoreMesh(core_axis_name="c", subcore_axis_name="s")

    @functools.partial(
        pl.kernel,
        mesh=mesh,
        compiler_params=pltpu.CompilerParams(use_tc_tiling_on_sc=False),
        out_type=[
            jax.ShapeDtypeStruct((NQ, NP, QD), jnp.float32),
            jax.ShapeDtypeStruct((NC, NP, QD), jnp.float32),
        ],
        scratch_types=[
            pltpu.VMEM((M, KB), jnp.int32),
            pltpu.VMEM((M, KB), jnp.int32),
            pltpu.VMEM((KB, QD), jnp.float32),
            pltpu.VMEM((KB, QD), jnp.float32),
            pltpu.VMEM_SHARED((NP, QD), jnp.float32),
            pltpu.SemaphoreType.DMA,
            pltpu.SemaphoreType.DMA,
        ],
    )
    def k(xq_h, gidx_h, ddst_h, oidx_h, eyeb_h, za_h,
          aggr_o, cnt_o, gidx_v, ddst_v, buf0, buf1, aggr_s, sem0, sem1):
        c = lax.axis_index("c")
        s = lax.axis_index("s")
        bufs = (buf0, buf1)
        sems = (sem0, sem1)

        def stream_phase(src_h, idx_of, count):
            pltpu.async_copy(src_h.at[gidx_v.at[idx_of(0)]], buf0, sem0)

            def body(i0, carry):
                for b in range(2):
                    i = i0 + b
                    m = idx_of(i)
                    pltpu.make_async_copy(
                        src_h.at[gidx_v.at[m]], bufs[b], sems[b]).wait()

                    @pl.when(i < count - 1)
                    def _():
                        pltpu.async_copy(
                            src_h.at[gidx_v.at[idx_of(i + 1)]], bufs[1 - b],
                            sems[1 - b])

                    pltpu.sync_copy(bufs[b], aggr_s.at[ddst_v.at[m]],
                                    add=True)
                return carry

            lax.fori_loop(0, count // 2, lambda i, cy: body(i * 2, cy), 0)

        for q in range(2):
            qi = 2 * q + c
            pltpu.sync_copy(gidx_h.at[qi, s], gidx_v)
            pltpu.sync_copy(ddst_h.at[s], ddst_v)
            pltpu.sync_copy(za_h, aggr_s.at[pl.ds(s * RPT, RPT)])
            plsc.subcore_barrier()
            stream_phase(xq_h, lambda i: i, M)
            plsc.subcore_barrier()
            pltpu.sync_copy(aggr_s.at[pl.ds(s * RPT, RPT)],
                            aggr_o.at[qi, pl.ds(s * RPT, RPT)])
            plsc.subcore_barrier()

        pltpu.sync_copy(oidx_h.at[s], gidx_v)
        pltpu.sync_copy(za_h, aggr_s.at[pl.ds(s * RPT, RPT)])
        plsc.subcore_barrier()
        stream_phase(eyeb_h, lambda i: 2 * i + c, M // 2)
        plsc.subcore_barrier()
        pltpu.sync_copy(aggr_s.at[pl.ds(s * RPT, RPT)],
                        cnt_o.at[c, pl.ds(s * RPT, RPT)])

    return k(xq4, gidx4, ddst, oidx, eyeb, za)


def _tc_mlp(aggr4, cnt2, x, cconst, embC64, W1, b1, W2, b2):
    R = 400
    G = N // R

    def body(a4_ref, cnt_ref, x_ref, cc_ref, emb_ref, w1_ref, b1_ref,
             w2_ref, b2_ref, o_ref):
        a = jnp.concatenate(
            [a4_ref[0], a4_ref[1], a4_ref[2], a4_ref[3]], axis=1)
        a = a + x_ref[...] + cc_ref[...]
        cnt = cnt_ref[0] + cnt_ref[1]
        a = a + jnp.dot(cnt, emb_ref[...], preferred_element_type=jnp.float32)
        h1 = jnp.dot(a, w1_ref[...], preferred_element_type=jnp.float32)
        h1 = jnp.maximum(h1 + b1_ref[...], 0.0)
        o_ref[...] = jnp.dot(h1, w2_ref[...],
                             preferred_element_type=jnp.float32) + b2_ref[...]

    return pl.pallas_call(
        body,
        grid=(G,),
        in_specs=[
            pl.BlockSpec((NQ, R, QD), lambda i: (0, i, 0)),
            pl.BlockSpec((NC, R, QD), lambda i: (0, i, 0)),
            pl.BlockSpec((R, D), lambda i: (i, 0)),
            pl.BlockSpec((1, D), lambda i: (0, 0)),
            pl.BlockSpec((QD, D), lambda i: (0, 0)),
            pl.BlockSpec((D, H), lambda i: (0, 0)),
            pl.BlockSpec((1, H), lambda i: (0, 0)),
            pl.BlockSpec((H, D), lambda i: (0, 0)),
            pl.BlockSpec((1, D), lambda i: (0, 0)),
        ],
        out_specs=pl.BlockSpec((R, D), lambda i: (i, 0)),
        out_shape=jax.ShapeDtypeStruct((N, D), jnp.float32),
    )(aggr4, cnt2, x, cconst, embC64, W1, b1, W2, b2)


def kernel(x, edge_index, edge_attr, E1, E2, W1, b1, W2, b2):
    src = edge_index[0].astype(jnp.int32)
    dst = edge_index[1].astype(jnp.int32)
    combo = (edge_attr[:, 0] * 3 + edge_attr[:, 1]).astype(jnp.int32)
    k9 = jnp.arange(9)
    embC = (E1[k9 // 3] + E2[k9 % 3]).astype(jnp.float32)
    embC64 = jnp.concatenate(
        [embC, jnp.zeros((QD - 9, D), jnp.float32)], axis=0)
    xq4 = jnp.concatenate([x[:, k * QD:(k + 1) * QD] for k in range(NQ)],
                          axis=0)
    pad = EP - E
    srcp = jnp.concatenate([src, jnp.zeros((pad,), jnp.int32)])
    dstp = jnp.concatenate([dst, jnp.full((pad,), DUMP, jnp.int32)])
    cmbp = jnp.concatenate([combo, jnp.zeros((pad,), jnp.int32)])
    gidx4 = jnp.stack([srcp + q * N for q in range(NQ)]).reshape(
        NQ, NT, M, KB)
    ddst = dstp.reshape(NT, M, KB)
    oidx = cmbp.reshape(NT, M, KB)
    eyeb = jnp.eye(QD, dtype=jnp.float32)
    za = jnp.zeros((RPT, QD), jnp.float32)
    cconst = (E1[4] + E2[0]).reshape(1, D)
    aggr4, cnt2 = _sc_aggregate(xq4, gidx4, ddst, oidx, eyeb, za)
    return _tc_mlp(aggr4, cnt2, x, cconst, embC64, W1,
                   b1.reshape(1, H), W2, b2.reshape(1, D))


# speedup vs baseline: 2.3744x; 2.3744x over previous
import functools

import jax
import jax.numpy as jnp
from jax import lax
from jax.experimental import pallas as pl
from jax.experimental.pallas import tpu as pltpu
from jax.experimental.pallas import tpu_sc as plsc

N, E, D, H = 10000, 160000, 256, 512
QD = 64
NQ = 4
NC = 2
NT = 16
EP = 163840
EPT = EP // NT
KB = 256
M = EPT // KB
NP = 10240
RPT = NP // NT
DUMP = 10100
REP = 256


def _sc_aggregate(xq4, gidx4, ddst, oidx, eyeb, za):
    mesh = plsc.VectorSubcoreMesh(core_axis_name="c", subcore_axis_name="s")

    @functools.partial(
        pl.kernel,
        mesh=mesh,
        compiler_params=pltpu.CompilerParams(use_tc_tiling_on_sc=False),
        out_type=[
            jax.ShapeDtypeStruct((NQ, NP, QD), jnp.float32),
            jax.ShapeDtypeStruct((NC, NP, QD), jnp.float32),
        ],
        scratch_types=[
            pltpu.VMEM((M, KB), jnp.int32),
            pltpu.VMEM((M, KB), jnp.int32),
            pltpu.VMEM((KB, QD), jnp.float32),
            pltpu.VMEM((KB, QD), jnp.float32),
            pltpu.VMEM_SHARED((NP, QD), jnp.float32),
            pltpu.SemaphoreType.DMA,
            pltpu.SemaphoreType.DMA,
        ],
    )
    def k(xq_h, gidx_h, ddst_h, oidx_h, eyeb_h, za_h,
          aggr_o, cnt_o, gidx_v, ddst_v, buf0, buf1, aggr_s, sem0, sem1):
        c = lax.axis_index("c")
        s = lax.axis_index("s")
        bufs = (buf0, buf1)
        sems = (sem0, sem1)

        def stream_phase(src_h, idx_of, count):
            pltpu.async_copy(src_h.at[gidx_v.at[idx_of(0)]], buf0, sem0)

            def body(i0, carry):
                for b in range(2):
                    i = i0 + b
                    m = idx_of(i)
                    pltpu.make_async_copy(
                        src_h.at[gidx_v.at[m]], bufs[b], sems[b]).wait()

                    @pl.when(i < count - 1)
                    def _():
                        pltpu.async_copy(
                            src_h.at[gidx_v.at[idx_of(i + 1)]], bufs[1 - b],
                            sems[1 - b])

                    pltpu.sync_copy(bufs[b], aggr_s.at[ddst_v.at[m]],
                                    add=True)
                return carry

            lax.fori_loop(0, count // 2, lambda i, cy: body(i * 2, cy), 0)

        for q in range(2):
            qi = 2 * q + c
            pltpu.sync_copy(gidx_h.at[qi, s], gidx_v)
            pltpu.sync_copy(ddst_h.at[s], ddst_v)
            pltpu.sync_copy(za_h, aggr_s.at[pl.ds(s * RPT, RPT)])
            plsc.subcore_barrier()
            stream_phase(xq_h, lambda i: i, M)
            plsc.subcore_barrier()
            pltpu.sync_copy(aggr_s.at[pl.ds(s * RPT, RPT)],
                            aggr_o.at[qi, pl.ds(s * RPT, RPT)])
            plsc.subcore_barrier()

        pltpu.sync_copy(oidx_h.at[s], gidx_v)
        pltpu.sync_copy(za_h, aggr_s.at[pl.ds(s * RPT, RPT)])
        plsc.subcore_barrier()
        stream_phase(eyeb_h, lambda i: 2 * i + c, M // 2)
        plsc.subcore_barrier()
        pltpu.sync_copy(aggr_s.at[pl.ds(s * RPT, RPT)],
                        cnt_o.at[c, pl.ds(s * RPT, RPT)])

    return k(xq4, gidx4, ddst, oidx, eyeb, za)


def _tc_mlp(aggr4, cnt2, x, cconst, embC64, W1, b1, W2, b2):
    R = 400
    G = N // R

    def body(a4_ref, cnt_ref, x_ref, cc_ref, emb_ref, w1_ref, b1_ref,
             w2_ref, b2_ref, o_ref):
        a = jnp.concatenate(
            [a4_ref[0], a4_ref[1], a4_ref[2], a4_ref[3]], axis=1)
        a = a + x_ref[...] + cc_ref[...]
        cnt = cnt_ref[0] + cnt_ref[1]
        a = a + jnp.dot(cnt, emb_ref[...], preferred_element_type=jnp.float32)
        h1 = jnp.dot(a, w1_ref[...], preferred_element_type=jnp.float32)
        h1 = jnp.maximum(h1 + b1_ref[...], 0.0)
        o_ref[...] = jnp.dot(h1, w2_ref[...],
                             preferred_element_type=jnp.float32) + b2_ref[...]

    return pl.pallas_call(
        body,
        grid=(G,),
        in_specs=[
            pl.BlockSpec((NQ, R, QD), lambda i: (0, i, 0)),
            pl.BlockSpec((NC, R, QD), lambda i: (0, i, 0)),
            pl.BlockSpec((R, D), lambda i: (i, 0)),
            pl.BlockSpec((1, D), lambda i: (0, 0)),
            pl.BlockSpec((QD, D), lambda i: (0, 0)),
            pl.BlockSpec((D, H), lambda i: (0, 0)),
            pl.BlockSpec((1, H), lambda i: (0, 0)),
            pl.BlockSpec((H, D), lambda i: (0, 0)),
            pl.BlockSpec((1, D), lambda i: (0, 0)),
        ],
        out_specs=pl.BlockSpec((R, D), lambda i: (i, 0)),
        out_shape=jax.ShapeDtypeStruct((N, D), jnp.float32),
    )(aggr4, cnt2, x, cconst, embC64, W1, b1, W2, b2)


def kernel(x, edge_index, edge_attr, E1, E2, W1, b1, W2, b2):
    src = edge_index[0].astype(jnp.int32)
    dst = edge_index[1].astype(jnp.int32)
    combo = (edge_attr[:, 0] * 3 + edge_attr[:, 1]).astype(jnp.int32)
    k9 = jnp.arange(9)
    embC = (E1[k9 // 3] + E2[k9 % 3]).astype(jnp.float32)
    embC64 = jnp.concatenate(
        [embC, jnp.zeros((QD - 9, D), jnp.float32)], axis=0)
    xq4 = jnp.concatenate([x[:, k * QD:(k + 1) * QD] for k in range(NQ)],
                          axis=0)
    pad = EP - E
    srcp = jnp.concatenate([src, jnp.zeros((pad,), jnp.int32)])
    dstp = jnp.concatenate([dst, jnp.full((pad,), DUMP, jnp.int32)])
    cmbp = jnp.concatenate([combo, jnp.zeros((pad,), jnp.int32)])
    gidx4 = jnp.stack([srcp + q * N for q in range(NQ)]).reshape(
        NQ, NT, M, KB)
    ddst = dstp.reshape(NT, M, KB)
    oidx = (cmbp * REP
            + (jnp.arange(EP, dtype=jnp.int32) % REP)).reshape(NT, M, KB)
    eyeb = jnp.repeat(jnp.eye(QD, dtype=jnp.float32)[:9], REP, axis=0)
    za = jnp.zeros((RPT, QD), jnp.float32)
    cconst = (E1[4] + E2[0]).reshape(1, D)
    aggr4, cnt2 = _sc_aggregate(xq4, gidx4, ddst, oidx, eyeb, za)
    return _tc_mlp(aggr4, cnt2, x, cconst, embC64, W1,
                   b1.reshape(1, H), W2, b2.reshape(1, D))


# speedup vs baseline: 2.4967x; 1.0515x over previous
import functools

import jax
import jax.numpy as jnp
from jax import lax
from jax.experimental import pallas as pl
from jax.experimental.pallas import tpu as pltpu
from jax.experimental.pallas import tpu_sc as plsc

N, E, D, H = 10000, 160000, 256, 512
QD = 64
NQ = 4
NC = 2
NT = 16
EP = 163840
EPT = EP // NT
KB = 256
M = EPT // KB
NP = 10240
RPT = NP // NT
DUMP = 10100
REP = 256


def _sc_aggregate(xq4, gidx4, ddst, oidx, eyeb, za):
    mesh = plsc.VectorSubcoreMesh(core_axis_name="c", subcore_axis_name="s")

    @functools.partial(
        pl.kernel,
        mesh=mesh,
        compiler_params=pltpu.CompilerParams(use_tc_tiling_on_sc=False),
        out_type=[
            jax.ShapeDtypeStruct((NQ, NP, QD), jnp.float32),
            jax.ShapeDtypeStruct((NC, NP, QD), jnp.float32),
        ],
        scratch_types=[
            pltpu.VMEM((M, KB), jnp.int32),
            pltpu.VMEM((M, KB), jnp.int32),
            pltpu.VMEM((KB, QD), jnp.float32),
            pltpu.VMEM((KB, QD), jnp.float32),
            pltpu.VMEM((KB, QD), jnp.float32),
            pltpu.VMEM((KB, QD), jnp.float32),
            pltpu.VMEM_SHARED((NP, QD), jnp.float32),
            pltpu.SemaphoreType.DMA,
            pltpu.SemaphoreType.DMA,
            pltpu.SemaphoreType.DMA,
            pltpu.SemaphoreType.DMA,
            pltpu.SemaphoreType.DMA,
            pltpu.SemaphoreType.DMA,
            pltpu.SemaphoreType.DMA,
            pltpu.SemaphoreType.DMA,
        ],
    )
    def k(xq_h, gidx_h, ddst_h, oidx_h, eyeb_h, za_h,
          aggr_o, cnt_o, gidx_v, ddst_v, buf0, buf1, buf2, buf3, aggr_s,
          g0, g1, g2, g3, s0, s1, s2, s3):
        c = lax.axis_index("c")
        s = lax.axis_index("s")
        bufs = (buf0, buf1, buf2, buf3)
        gsems = (g0, g1, g2, g3)
        ssems = (s0, s1, s2, s3)

        def stream_phase(src_h, idx_of, count):
            pltpu.async_copy(src_h.at[gidx_v.at[idx_of(0)]], bufs[0],
                             gsems[0])
            pltpu.async_copy(src_h.at[gidx_v.at[idx_of(1)]], bufs[1],
                             gsems[1])

            def body(i0, carry):
                for b4 in range(4):
                    i = i0 + b4
                    slot = b4 & 3
                    nslot = (b4 + 2) & 3

                    @pl.when(i >= 2)
                    def _drain():
                        pltpu.make_async_copy(
                            bufs[nslot],
                            aggr_s.at[ddst_v.at[idx_of(i - 2)]],
                            ssems[nslot]).wait()

                    @pl.when(i + 2 < count)
                    def _prefetch():
                        pltpu.async_copy(
                            src_h.at[gidx_v.at[idx_of(i + 2)]], bufs[nslot],
                            gsems[nslot])

                    pltpu.make_async_copy(
                        src_h.at[gidx_v.at[idx_of(i)]], bufs[slot],
                        gsems[slot]).wait()
                    pltpu.async_copy(bufs[slot],
                                     aggr_s.at[ddst_v.at[idx_of(i)]],
                                     ssems[slot], add=True)
                return carry

            lax.fori_loop(0, count // 4, lambda i, cy: body(i * 4, cy), 0)
            for i in (count - 2, count - 1):
                pltpu.make_async_copy(
                    bufs[i % 4], aggr_s.at[ddst_v.at[idx_of(i)]],
                    ssems[i % 4]).wait()

        for q in range(2):
            qi = 2 * q + c
            pltpu.sync_copy(gidx_h.at[qi, s], gidx_v)
            pltpu.sync_copy(ddst_h.at[s], ddst_v)
            pltpu.sync_copy(za_h, aggr_s.at[pl.ds(s * RPT, RPT)])
            plsc.subcore_barrier()
            stream_phase(xq_h, lambda i: i, M)
            plsc.subcore_barrier()
            pltpu.sync_copy(aggr_s.at[pl.ds(s * RPT, RPT)],
                            aggr_o.at[qi, pl.ds(s * RPT, RPT)])
            plsc.subcore_barrier()

        pltpu.sync_copy(oidx_h.at[s], gidx_v)
        pltpu.sync_copy(za_h, aggr_s.at[pl.ds(s * RPT, RPT)])
        plsc.subcore_barrier()
        stream_phase(eyeb_h, lambda i: 2 * i + c, M // 2)
        plsc.subcore_barrier()
        pltpu.sync_copy(aggr_s.at[pl.ds(s * RPT, RPT)],
                        cnt_o.at[c, pl.ds(s * RPT, RPT)])

    return k(xq4, gidx4, ddst, oidx, eyeb, za)


def _tc_mlp(aggr4, cnt2, x, cconst, embC64, W1, b1, W2, b2):
    R = 400
    G = N // R

    def body(a4_ref, cnt_ref, x_ref, cc_ref, emb_ref, w1_ref, b1_ref,
             w2_ref, b2_ref, o_ref):
        a = jnp.concatenate(
            [a4_ref[0], a4_ref[1], a4_ref[2], a4_ref[3]], axis=1)
        a = a + x_ref[...] + cc_ref[...]
        cnt = cnt_ref[0] + cnt_ref[1]
        a = a + jnp.dot(cnt, emb_ref[...], preferred_element_type=jnp.float32)
        h1 = jnp.dot(a, w1_ref[...], preferred_element_type=jnp.float32)
        h1 = jnp.maximum(h1 + b1_ref[...], 0.0)
        o_ref[...] = jnp.dot(h1, w2_ref[...],
                             preferred_element_type=jnp.float32) + b2_ref[...]

    return pl.pallas_call(
        body,
        grid=(G,),
        in_specs=[
            pl.BlockSpec((NQ, R, QD), lambda i: (0, i, 0)),
            pl.BlockSpec((NC, R, QD), lambda i: (0, i, 0)),
            pl.BlockSpec((R, D), lambda i: (i, 0)),
            pl.BlockSpec((1, D), lambda i: (0, 0)),
            pl.BlockSpec((QD, D), lambda i: (0, 0)),
            pl.BlockSpec((D, H), lambda i: (0, 0)),
            pl.BlockSpec((1, H), lambda i: (0, 0)),
            pl.BlockSpec((H, D), lambda i: (0, 0)),
            pl.BlockSpec((1, D), lambda i: (0, 0)),
        ],
        out_specs=pl.BlockSpec((R, D), lambda i: (i, 0)),
        out_shape=jax.ShapeDtypeStruct((N, D), jnp.float32),
    )(aggr4, cnt2, x, cconst, embC64, W1, b1, W2, b2)


def kernel(x, edge_index, edge_attr, E1, E2, W1, b1, W2, b2):
    src = edge_index[0].astype(jnp.int32)
    dst = edge_index[1].astype(jnp.int32)
    combo = (edge_attr[:, 0] * 3 + edge_attr[:, 1]).astype(jnp.int32)
    k9 = jnp.arange(9)
    embC = (E1[k9 // 3] + E2[k9 % 3]).astype(jnp.float32)
    embC64 = jnp.concatenate(
        [embC, jnp.zeros((QD - 9, D), jnp.float32)], axis=0)
    xq4 = jnp.concatenate([x[:, k * QD:(k + 1) * QD] for k in range(NQ)],
                          axis=0)
    pad = EP - E
    srcp = jnp.concatenate([src, jnp.zeros((pad,), jnp.int32)])
    dstp = jnp.concatenate([dst, jnp.full((pad,), DUMP, jnp.int32)])
    cmbp = jnp.concatenate([combo, jnp.zeros((pad,), jnp.int32)])
    gidx4 = jnp.stack([srcp + q * N for q in range(NQ)]).reshape(
        NQ, NT, M, KB)
    ddst = dstp.reshape(NT, M, KB)
    oidx = (cmbp * REP
            + (jnp.arange(EP, dtype=jnp.int32) % REP)).reshape(NT, M, KB)
    eyeb = jnp.repeat(jnp.eye(QD, dtype=jnp.float32)[:9], REP, axis=0)
    za = jnp.zeros((RPT, QD), jnp.float32)
    cconst = (E1[4] + E2[0]).reshape(1, D)
    aggr4, cnt2 = _sc_aggregate(xq4, gidx4, ddst, oidx, eyeb, za)
    return _tc_mlp(aggr4, cnt2, x, cconst, embC64, W1,
                   b1.reshape(1, H), W2, b2.reshape(1, D))


# speedup vs baseline: 3.1727x; 1.2707x over previous
import functools

import jax
import jax.numpy as jnp
from jax import lax
from jax.experimental import pallas as pl
from jax.experimental.pallas import tpu as pltpu
from jax.experimental.pallas import tpu_sc as plsc

N, E, D, H = 10000, 160000, 256, 512
QD = 64
NQ = 4
NC = 2
NT = 16
EP = 163840
EPT = EP // NT
KB = 256
M = EPT // KB
NP = 10240
RPT = NP // NT
DUMP = 10100
REP = 256


def _sc_aggregate(xq4, gidx4, ddst, oidx, eyeb, za, zc):
    mesh = plsc.VectorSubcoreMesh(core_axis_name="c", subcore_axis_name="s")

    @functools.partial(
        pl.kernel,
        mesh=mesh,
        compiler_params=pltpu.CompilerParams(use_tc_tiling_on_sc=False),
        out_type=[
            jax.ShapeDtypeStruct((NQ, NP, QD), jnp.bfloat16),
            jax.ShapeDtypeStruct((NC, NP, 16), jnp.float32),
        ],
        scratch_types=[
            pltpu.VMEM((M, KB), jnp.int32),
            pltpu.VMEM((M, KB), jnp.int32),
            pltpu.VMEM((KB, QD), jnp.bfloat16),
            pltpu.VMEM((KB, QD), jnp.bfloat16),
            pltpu.VMEM((KB, QD), jnp.bfloat16),
            pltpu.VMEM((KB, QD), jnp.bfloat16),
            pltpu.VMEM((KB, 16), jnp.float32),
            pltpu.VMEM((KB, 16), jnp.float32),
            pltpu.VMEM((KB, 16), jnp.float32),
            pltpu.VMEM((KB, 16), jnp.float32),
            pltpu.VMEM_SHARED((NP, QD), jnp.bfloat16),
            pltpu.VMEM_SHARED((NP, 16), jnp.float32),
            pltpu.SemaphoreType.DMA,
            pltpu.SemaphoreType.DMA,
            pltpu.SemaphoreType.DMA,
            pltpu.SemaphoreType.DMA,
            pltpu.SemaphoreType.DMA,
            pltpu.SemaphoreType.DMA,
            pltpu.SemaphoreType.DMA,
            pltpu.SemaphoreType.DMA,
        ],
    )
    def k(xq_h, gidx_h, ddst_h, oidx_h, eyeb_h, za_h, zc_h,
          aggr_o, cnt_o, gidx_v, ddst_v, buf0, buf1, buf2, buf3,
          cb0, cb1, cb2, cb3, aggr_s, cnt_s,
          g0, g1, g2, g3, s0, s1, s2, s3):
        c = lax.axis_index("c")
        s = lax.axis_index("s")
        xbufs = (buf0, buf1, buf2, buf3)
        cbufs = (cb0, cb1, cb2, cb3)
        gsems = (g0, g1, g2, g3)
        ssems = (s0, s1, s2, s3)

        def stream_phase(src_h, idx_of, count, bufs, tgt_s):
            pltpu.async_copy(src_h.at[gidx_v.at[idx_of(0)]], bufs[0],
                             gsems[0])
            pltpu.async_copy(src_h.at[gidx_v.at[idx_of(1)]], bufs[1],
                             gsems[1])

            def body(i0, carry):
                for b4 in range(4):
                    i = i0 + b4
                    slot = b4 & 3
                    nslot = (b4 + 2) & 3

                    @pl.when(i >= 2)
                    def _drain():
                        pltpu.make_async_copy(
                            bufs[nslot],
                            tgt_s.at[ddst_v.at[idx_of(i - 2)]],
                            ssems[nslot]).wait()

                    @pl.when(i + 2 < count)
                    def _prefetch():
                        pltpu.async_copy(
                            src_h.at[gidx_v.at[idx_of(i + 2)]], bufs[nslot],
                            gsems[nslot])

                    pltpu.make_async_copy(
                        src_h.at[gidx_v.at[idx_of(i)]], bufs[slot],
                        gsems[slot]).wait()
                    pltpu.async_copy(bufs[slot],
                                     tgt_s.at[ddst_v.at[idx_of(i)]],
                                     ssems[slot], add=True)
                return carry

            lax.fori_loop(0, count // 4, lambda i, cy: body(i * 4, cy), 0)
            for i in (count - 2, count - 1):
                pltpu.make_async_copy(
                    bufs[i % 4], tgt_s.at[ddst_v.at[idx_of(i)]],
                    ssems[i % 4]).wait()

        for q in range(2):
            qi = 2 * q + c
            pltpu.sync_copy(gidx_h.at[qi, s], gidx_v)
            pltpu.sync_copy(ddst_h.at[s], ddst_v)
            pltpu.sync_copy(za_h, aggr_s.at[pl.ds(s * RPT, RPT)])
            plsc.subcore_barrier()
            stream_phase(xq_h, lambda i: i, M, xbufs, aggr_s)
            plsc.subcore_barrier()
            pltpu.sync_copy(aggr_s.at[pl.ds(s * RPT, RPT)],
                            aggr_o.at[qi, pl.ds(s * RPT, RPT)])
            plsc.subcore_barrier()

        pltpu.sync_copy(oidx_h.at[s], gidx_v)
        pltpu.sync_copy(zc_h, cnt_s.at[pl.ds(s * RPT, RPT)])
        plsc.subcore_barrier()
        stream_phase(eyeb_h, lambda i: 2 * i + c, M // 2, cbufs, cnt_s)
        plsc.subcore_barrier()
        pltpu.sync_copy(cnt_s.at[pl.ds(s * RPT, RPT)],
                        cnt_o.at[c, pl.ds(s * RPT, RPT)])

    return k(xq4, gidx4, ddst, oidx, eyeb, za, zc)


def _tc_mlp(aggr4, cnt2, x, cconst, embC16, W1, b1, W2, b2):
    R = 400
    G = N // R

    def body(a4_ref, cnt_ref, x_ref, cc_ref, emb_ref, w1_ref, b1_ref,
             w2_ref, b2_ref, o_ref):
        a = jnp.concatenate(
            [a4_ref[0], a4_ref[1], a4_ref[2], a4_ref[3]],
            axis=1).astype(jnp.float32)
        a = a + x_ref[...] + cc_ref[...]
        cnt = cnt_ref[0] + cnt_ref[1]
        a = a + jnp.dot(cnt, emb_ref[...], preferred_element_type=jnp.float32)
        h1 = jnp.dot(a, w1_ref[...], preferred_element_type=jnp.float32)
        h1 = jnp.maximum(h1 + b1_ref[...], 0.0)
        o_ref[...] = jnp.dot(h1, w2_ref[...],
                             preferred_element_type=jnp.float32) + b2_ref[...]

    return pl.pallas_call(
        body,
        grid=(G,),
        in_specs=[
            pl.BlockSpec((NQ, R, QD), lambda i: (0, i, 0)),
            pl.BlockSpec((NC, R, 16), lambda i: (0, i, 0)),
            pl.BlockSpec((R, D), lambda i: (i, 0)),
            pl.BlockSpec((1, D), lambda i: (0, 0)),
            pl.BlockSpec((16, D), lambda i: (0, 0)),
            pl.BlockSpec((D, H), lambda i: (0, 0)),
            pl.BlockSpec((1, H), lambda i: (0, 0)),
            pl.BlockSpec((H, D), lambda i: (0, 0)),
            pl.BlockSpec((1, D), lambda i: (0, 0)),
        ],
        out_specs=pl.BlockSpec((R, D), lambda i: (i, 0)),
        out_shape=jax.ShapeDtypeStruct((N, D), jnp.float32),
    )(aggr4, cnt2, x, cconst, embC16, W1, b1, W2, b2)


def kernel(x, edge_index, edge_attr, E1, E2, W1, b1, W2, b2):
    src = edge_index[0].astype(jnp.int32)
    dst = edge_index[1].astype(jnp.int32)
    combo = (edge_attr[:, 0] * 3 + edge_attr[:, 1]).astype(jnp.int32)
    k9 = jnp.arange(9)
    embC = (E1[k9 // 3] + E2[k9 % 3]).astype(jnp.float32)
    embC16 = jnp.concatenate(
        [embC, jnp.zeros((16 - 9, D), jnp.float32)], axis=0)
    xq4 = jnp.concatenate([x[:, k * QD:(k + 1) * QD] for k in range(NQ)],
                          axis=0).astype(jnp.bfloat16)
    pad = EP - E
    srcp = jnp.concatenate([src, jnp.zeros((pad,), jnp.int32)])
    dstp = jnp.concatenate([dst, jnp.full((pad,), DUMP, jnp.int32)])
    cmbp = jnp.concatenate([combo, jnp.zeros((pad,), jnp.int32)])
    gidx4 = jnp.stack([srcp + q * N for q in range(NQ)]).reshape(
        NQ, NT, M, KB)
    ddst = dstp.reshape(NT, M, KB)
    oidx = (cmbp * REP
            + (jnp.arange(EP, dtype=jnp.int32) % REP)).reshape(NT, M, KB)
    eyeb = jnp.repeat(jnp.eye(16, dtype=jnp.float32)[:9], REP, axis=0)
    za = jnp.zeros((RPT, QD), jnp.bfloat16)
    zc = jnp.zeros((RPT, 16), jnp.float32)
    cconst = (E1[4] + E2[0]).reshape(1, D)
    aggr4, cnt2 = _sc_aggregate(xq4, gidx4, ddst, oidx, eyeb, za, zc)
    return _tc_mlp(aggr4, cnt2, x, cconst, embC16, W1,
                   b1.reshape(1, H), W2, b2.reshape(1, D))
